# baseline (speedup 1.0000x reference)
import jax
import jax.numpy as jnp
from jax import lax
from jax.experimental import pallas as pl
from jax.experimental.pallas import tpu as pltpu
from jax.experimental.pallas import tpu_sc as plsc

_N = 10000
_E = 320000
_ELG = 640000
_D = 128

_NC = 2
_NS = 16
_NP = 10240

_SC_MESH = dict(core_axis_name="c", subcore_axis_name="s",
                num_cores=_NC, num_subcores=_NS)


def _spmm_dual_body(x2_hbm, u_hbm, v_hbm, zeros_hbm, out_hbm,
                    idx_u, idx_v, idx_ua, idx_va, rows_a, rows_b,
                    acc, sem_a, sem_b):
    c = lax.axis_index("c")
    s = lax.axis_index("s")
    rows_per_sub = _NP // _NS
    pltpu.sync_copy(zeros_hbm.at[pl.ds(s * rows_per_sub, rows_per_sub)],
                    acc.at[pl.ds(s * rows_per_sub, rows_per_sub)])
    plsc.subcore_barrier()

    ch = 80
    chunks = _E // (_NS * ch)
    base0 = s * (_E // _NS)
    coff = c * _NP

    def body(g, carry):
        base = base0 + g * ch
        pltpu.sync_copy(u_hbm.at[pl.ds(base, ch)], idx_u)
        pltpu.sync_copy(v_hbm.at[pl.ds(base, ch)], idx_v)
        for j in range(ch // 16):
            sl = pl.ds(j * 16, 16)
            idx_ua[sl] = idx_u[sl] + coff
            idx_va[sl] = idx_v[sl] + coff
        ga = pltpu.async_copy(x2_hbm.at[idx_ua], rows_a, sem_a)
        gb = pltpu.async_copy(x2_hbm.at[idx_va], rows_b, sem_b)
        ga.wait()
        gb.wait()
        pltpu.sync_copy(rows_a, acc.at[idx_v], add=True)
        pltpu.sync_copy(rows_b, acc.at[idx_u], add=True)
        return carry

    lax.fori_loop(0, chunks, body, 0)
    plsc.subcore_barrier()
    pltpu.sync_copy(acc.at[pl.ds(s * rows_per_sub, rows_per_sub)],
                    out_hbm.at[c, pl.ds(s * rows_per_sub, rows_per_sub)])


def _spmm_dual(xa, xb, u, v):
    x2 = jnp.zeros((2 * _NP, _D), jnp.float32)
    x2 = x2.at[:_N].set(xa).at[_NP:_NP + _N].set(xb)
    zeros = jnp.zeros((_NP, _D), jnp.float32)
    mesh = plsc.VectorSubcoreMesh(**_SC_MESH)
    ch = 80
    k = pl.kernel(
        _spmm_dual_body,
        out_type=jax.ShapeDtypeStruct((_NC, _NP, _D), jnp.float32),
        mesh=mesh,
        scratch_types=[
            pltpu.VMEM((ch,), jnp.int32),
            pltpu.VMEM((ch,), jnp.int32),
            pltpu.VMEM((ch,), jnp.int32),
            pltpu.VMEM((ch,), jnp.int32),
            pltpu.VMEM((ch, _D), jnp.float32),
            pltpu.VMEM((ch, _D), jnp.float32),
            pltpu.VMEM_SHARED((_NP, _D), jnp.float32),
            pltpu.SemaphoreType.DMA,
            pltpu.SemaphoreType.DMA,
        ],
    )
    out = k(x2, u, v, zeros)
    return out[0, :_N], out[1, :_N]


def _elu(t):
    return jnp.where(t > 0, t, jnp.exp(t) - 1.0)


_BLK = 2000


def _prologue_body(x_ref, mfeat_ref, atom_ref, mW_ref, mb_ref, cnt8_ref,
                   bond8_ref, deg_ref,
                   n1_ref, m1_ref, xsa_ref, xsb_ref, da_ref, db_ref,
                   nrm_ref, ivd_ref):
    cols = lax.broadcasted_iota(jnp.int32, (_BLK, _D), 1)
    oh = jnp.where(cols == x_ref[...], 1.0, 0.0)
    n1 = jnp.dot(oh, atom_ref[...], preferred_element_type=jnp.float32)
    m1 = jnp.dot(mfeat_ref[...], mW_ref[...],
                 preferred_element_type=jnp.float32) + mb_ref[...]
    Te = jnp.dot(cnt8_ref[...], bond8_ref[...],
                 preferred_element_type=jnp.float32)
    nf = _elu(n1 * Te)
    mf = _elu(m1 * Te)
    degp = deg_ref[...] + 1.0
    nrm = lax.rsqrt(degp)
    ivd = 1.0 / degp
    n1_ref[...] = n1
    m1_ref[...] = m1
    xsa_ref[...] = nf * nrm
    xsb_ref[...] = mf * nrm
    da_ref[...] = nf * ivd
    db_ref[...] = mf * ivd
    nrm_ref[...] = nrm
    ivd_ref[...] = ivd


def _prologue(x, metafeat, atom_table, meta_W, meta_b, cnt, bond_table, deg):
    x2d = x.reshape(_N, 1).astype(jnp.int32)
    atom128 = jnp.zeros((_D, _D), jnp.float32).at[:119].set(atom_table)
    bond8 = jnp.zeros((8, _D), jnp.float32).at[:5].set(bond_table)
    cnt8 = jnp.zeros((_N, 8), jnp.float32).at[:, :5].set(cnt.reshape(_N, 5))
    nd = jax.ShapeDtypeStruct((_N, _D), jnp.float32)
    n1 = jax.ShapeDtypeStruct((_N, 1), jnp.float32)
    row = lambda w: pl.BlockSpec((_BLK, w), lambda i: (i, 0))
    full = lambda a, b: pl.BlockSpec((a, b), lambda i: (0, 0))
    return pl.pallas_call(
        _prologue_body,
        grid=(_N // _BLK,),
        in_specs=[row(1), row(16), full(_D, _D), full(16, _D), full(1, _D),
                  row(8), full(8, _D), row(1)],
        out_specs=(row(_D), row(_D), row(_D), row(_D), row(_D), row(_D),
                   row(1), row(1)),
        out_shape=(nd, nd, nd, nd, nd, nd, n1, n1),
    )(x2d, metafeat, atom128, meta_W, meta_b.reshape(1, _D), cnt8, bond8,
      deg.reshape(_N, 1))


def _mid_body(ya_ref, yb_ref, da_ref, db_ref, Wo_ref, bo_ref, Wm_ref, bm_ref,
              P_ref, Wlg_ref, blg_ref, deg_ref, nrm_ref, ivd_ref,
              x2a_ref, x2b_ref, d2a_ref, d2b_ref):
    nrm = nrm_ref[...]
    ivd = ivd_ref[...]
    h_org = jnp.dot(ya_ref[...] * nrm + da_ref[...], Wo_ref[...],
                    preferred_element_type=jnp.float32) + bo_ref[...]
    h_meta = jnp.dot(yb_ref[...] * nrm + db_ref[...], Wm_ref[...],
                     preferred_element_type=jnp.float32) + bm_ref[...]
    Th = jnp.dot(P_ref[...], Wlg_ref[...],
                 preferred_element_type=jnp.float32) + deg_ref[...] * blg_ref[...]
    a_o = _elu(h_org * Th)
    a_m = _elu(h_meta * Th)
    x2a_ref[...] = a_o * nrm
    x2b_ref[...] = a_m * nrm
    d2a_ref[...] = a_o * ivd
    d2b_ref[...] = a_m * ivd


def _mid(ya, yb, da, db, W_org, b_org, W_meta, b_meta, P, W_lg, b_lg, deg,
         nrm, ivd):
    nd = jax.ShapeDtypeStruct((_N, _D), jnp.float32)
    row = lambda w: pl.BlockSpec((_BLK, w), lambda i: (i, 0))
    full = lambda a, b: pl.BlockSpec((a, b), lambda i: (0, 0))
    return pl.pallas_call(
        _mid_body,
        grid=(_N // _BLK,),
        in_specs=[row(_D), row(_D), row(_D), row(_D), full(_D, _D),
                  full(1, _D), full(_D, _D), full(1, _D), row(_D),
                  full(_D, _D), full(1, _D), row(1), row(1), row(1)],
        out_specs=(row(_D), row(_D), row(_D), row(_D)),
        out_shape=(nd, nd, nd, nd),
    )(ya, yb, da, db, W_org, b_org.reshape(1, _D), W_meta,
      b_meta.reshape(1, _D), P, W_lg, b_lg.reshape(1, _D),
      deg.reshape(_N, 1), nrm, ivd)


_CNT0 = 61440
_DEGOFF = 51200
_CNT1 = 321536
_LGROWS = 10240


def _counts_body(u2f_hbm, v2f_hbm, t2f_hbm, lg2f_hbm, out0_hbm, out1_hbm,
                 u2d, v2d, t2d, idx4, ones, zbuf, acc, sem_sc):
    c = lax.axis_index("c")
    s = lax.axis_index("s")
    rows = _EROWS // _NS
    lgrows = _LGROWS // _NS

    for j in range(128 // 16):
        sl = pl.ds(j * 16, 16)
        ones[sl] = jnp.ones((16,), jnp.float32)
    for j in range(2048 // 16):
        sl = pl.ds(j * 16, 16)
        zbuf[sl] = jnp.zeros((16,), jnp.float32)

    @pl.when(c == 0)
    def _():
        pltpu.sync_copy(zbuf, acc.at[pl.ds(s * 3840, 2048)])
        pltpu.sync_copy(zbuf.at[pl.ds(0, 1792)],
                        acc.at[pl.ds(s * 3840 + 2048, 1792)])

    @pl.when(c == 1)
    def _():
        def z(j, carry):
            pltpu.sync_copy(zbuf, acc.at[pl.ds(s * 20096 + j * 2048, 2048)])
            return carry

        lax.fori_loop(0, 9, z, 0)
        pltpu.sync_copy(zbuf.at[pl.ds(0, 1664)],
                        acc.at[pl.ds(s * 20096 + 9 * 2048, 1664)])

    plsc.subcore_barrier()

    @pl.when(c == 0)
    def _():
        def ldrow(j, carry):
            base = (s * rows + j) * 128
            pltpu.sync_copy(u2f_hbm.at[pl.ds(base, 128)], u2d.at[j])
            pltpu.sync_copy(v2f_hbm.at[pl.ds(base, 128)], v2d.at[j])
            pltpu.sync_copy(t2f_hbm.at[pl.ds(base, 128)], t2d.at[j])
            return carry

        lax.fori_loop(0, rows, ldrow, 0)

        def row(j, carry):
            for i in range(8):
                sl = pl.ds(i * 16, 16)
                iu = u2d[j, sl]
                iv = v2d[j, sl]
                it = t2d[j, sl]
                idx4[0, sl] = iu * 5 + it
                idx4[1, sl] = iv * 5 + it
                idx4[2, sl] = iu + _DEGOFF
                idx4[3, sl] = iv + _DEGOFF
            for r in range(4):
                pltpu.sync_copy(ones, acc.at[idx4.at[r]], add=True)
            return carry

        lax.fori_loop(0, rows, row, 0)

    @pl.when(c == 1)
    def _():
        def row(j, carry):
            pltpu.sync_copy(lg2f_hbm.at[pl.ds((s * lgrows + j) * 128, 128)],
                            idx4.at[0])
            pltpu.sync_copy(ones, acc.at[idx4.at[0]], add=True)
            return carry

        lax.fori_loop(0, lgrows, row, 0)

    plsc.subcore_barrier()

    @pl.when(c == 0)
    def _():
        pltpu.sync_copy(acc.at[pl.ds(s * 3840, 3840)],
                        out0_hbm.at[pl.ds(s * 3840, 3840)])

    @pl.when(c == 1)
    def _():
        def wb(j, carry):
            pltpu.sync_copy(acc.at[pl.ds(s * 20096 + j * 2048, 2048)],
                            out1_hbm.at[pl.ds(s * 20096 + j * 2048, 2048)])
            return carry

        lax.fori_loop(0, 9, wb, 0)
        pltpu.sync_copy(acc.at[pl.ds(s * 20096 + 9 * 2048, 1664)],
                        out1_hbm.at[pl.ds(s * 20096 + 9 * 2048, 1664)])


def _counts(u2f, v2f, t2f, lg2f):
    mesh = plsc.VectorSubcoreMesh(**_SC_MESH)
    k = pl.kernel(
        _counts_body,
        out_type=(jax.ShapeDtypeStruct((_CNT0,), jnp.float32),
                  jax.ShapeDtypeStruct((_CNT1,), jnp.float32)),
        mesh=mesh,
        scratch_types=[
            pltpu.VMEM((_EROWS // _NS, 128), jnp.int32),
            pltpu.VMEM((_EROWS // _NS, 128), jnp.int32),
            pltpu.VMEM((_EROWS // _NS, 128), jnp.int32),
            pltpu.VMEM((4, 128), jnp.int32),
            pltpu.VMEM((128,), jnp.float32),
            pltpu.VMEM((2048,), jnp.float32),
            pltpu.VMEM_SHARED((_CNT1,), jnp.float32),
            pltpu.SemaphoreType.DMA,
        ],
    )
    return k(u2f, v2f, t2f, lg2f)


def _edge_build_body(nm_hbm, pk_hbm, nl_hbm, bt_hbm,
                     ef_hbm,
                     pbuf, nlv, idx_u, idx_v, bt_v, r_nm_u, r_nm_v, obuf,
                     sem0, sem1):
    c = lax.axis_index("c")
    s = lax.axis_index("s")
    w = c * _NS + s
    ch = 80
    per_w = _E // (_NC * _NS)
    chunks = per_w // ch
    base0 = w * per_w
    pltpu.sync_copy(bt_hbm, bt_v)

    def body(g, carry):
        base = base0 + g * ch
        pltpu.sync_copy(pk_hbm.at[pl.ds(base * 3, 3 * ch)], pbuf)
        pltpu.sync_copy(nl_hbm.at[pl.ds(base, ch)], nlv)
        for j in range(ch // 16):
            sl = pl.ds(j * 16, 16)
            idx_u[sl] = pbuf[sl]
            idx_v[sl] = pbuf[pl.ds(ch + j * 16, 16)]
        g0 = pltpu.async_copy(nm_hbm.at[idx_u], r_nm_u, sem0)
        g1 = pltpu.async_copy(nm_hbm.at[idx_v], r_nm_v, sem1)
        g0.wait()
        g1.wait()

        def rgrp(g2, carry2):
            tvec = pbuf[pl.ds(2 * ch + g2 * 16, 16)]
            nvec = nlv[pl.ds(g2 * 16, 16)]
            for i in range(16):
                r = g2 * 16 + i
                tr = tvec[i]
                nlr = nvec[i]
                for j in range(_D // 16):
                    sl = pl.ds(j * 16, 16)
                    sl2 = pl.ds(_D + j * 16, 16)
                    su = r_nm_u[r, sl] + r_nm_v[r, sl]
                    sm = r_nm_u[r, sl2] + r_nm_v[r, sl2]
                    btj = bt_v[pl.ds(tr * _D + j * 16, 16)]
                    val = btj * su * sm
                    val = jnp.where(val > 0, val, jnp.exp(val) - 1.0)
                    obuf[r, sl] = val * nlr
            return carry2

        lax.fori_loop(0, ch // 16, rgrp, 0)
        pltpu.sync_copy(obuf, ef_hbm.at[pl.ds(base, ch)])
        return carry

    lax.fori_loop(0, chunks, body, 0)


def _edge_build(n1, m1, u, v, t, norm_lg, bond_table):
    mesh = plsc.VectorSubcoreMesh(**_SC_MESH)
    ch = 80
    nm = jnp.concatenate([n1, m1], axis=1)
    pk = jnp.concatenate(
        [u.reshape(-1, ch), v.reshape(-1, ch), t.reshape(-1, ch)],
        axis=1).reshape(-1)
    k = pl.kernel(
        _edge_build_body,
        out_type=jax.ShapeDtypeStruct((_E, _D), jnp.float32),
        mesh=mesh,
        scratch_types=[
            pltpu.VMEM((3 * ch,), jnp.int32),
            pltpu.VMEM((ch,), jnp.float32),
            pltpu.VMEM((ch,), jnp.int32),
            pltpu.VMEM((ch,), jnp.int32),
            pltpu.VMEM((5 * _D,), jnp.float32),
            pltpu.VMEM((ch, 2 * _D), jnp.float32),
            pltpu.VMEM((ch, 2 * _D), jnp.float32),
            pltpu.VMEM((ch, _D), jnp.float32),
            pltpu.SemaphoreType.DMA,
            pltpu.SemaphoreType.DMA,
        ],
    )
    return k(nm, pk, norm_lg, bond_table.reshape(5 * _D))


def _lg_fused_body(ef_hbm, pk_hbm, u_hbm, v_hbm, nl_hbm, zeros_hbm,
                   out_hbm,
                   pbuf, idx_a, idx_b, idx_u, idx_v, sclv, rows,
                   acc, sem0, sem1, sem2, sem3):
    c = lax.axis_index("c")
    s = lax.axis_index("s")
    w = c * _NS + s
    rows_per_sub = _NP // _NS
    pltpu.sync_copy(zeros_hbm.at[pl.ds(s * rows_per_sub, rows_per_sub)],
                    acc.at[pl.ds(s * rows_per_sub, rows_per_sub)])
    plsc.subcore_barrier()

    ch = 80
    m_tot = 2 * _ELG + _E
    per_w = m_tot // (_NC * _NS)
    chunks = per_w // ch
    base0 = w * per_w

    def body(g, carry):
        base = base0 + g * ch
        pltpu.sync_copy(pk_hbm.at[pl.ds(base * 2, 2 * ch)], pbuf)
        for j in range(ch // 16):
            sl = pl.ds(j * 16, 16)
            idx_a[sl] = pbuf[sl]
            idx_b[sl] = pbuf[pl.ds(ch + j * 16, 16)]
        gr = pltpu.async_copy(ef_hbm.at[idx_a], rows, sem0)
        gs = pltpu.async_copy(nl_hbm.at[idx_b], sclv, sem1)
        gu = pltpu.async_copy(u_hbm.at[idx_b], idx_u, sem2)
        gv = pltpu.async_copy(v_hbm.at[idx_b], idx_v, sem3)
        gr.wait()
        gs.wait()
        gu.wait()
        gv.wait()

        def rgrp(g2, carry2):
            svec = sclv[pl.ds(g2 * 16, 16)]
            for i in range(16):
                r = g2 * 16 + i
                scr = svec[i]
                for j in range(_D // 16):
                    sl = pl.ds(j * 16, 16)
                    rows[r, sl] = rows[r, sl] * scr
            return carry2

        lax.fori_loop(0, ch // 16, rgrp, 0)
        pltpu.sync_copy(rows, acc.at[idx_u], add=True)
        pltpu.sync_copy(rows, acc.at[idx_v], add=True)
        return carry

    lax.fori_loop(0, chunks, body, 0)
    plsc.subcore_barrier()
    pltpu.sync_copy(acc.at[pl.ds(s * rows_per_sub, rows_per_sub)],
                    out_hbm.at[c, pl.ds(s * rows_per_sub, rows_per_sub)])


def _lg_fused(ef, src_all, dst_all, u, v, norm_lg):
    zeros = jnp.zeros((_NP, _D), jnp.float32)
    mesh = plsc.VectorSubcoreMesh(**_SC_MESH)
    ch = 80
    pk = jnp.concatenate(
        [src_all.reshape(-1, ch), dst_all.reshape(-1, ch)],
        axis=1).reshape(-1)
    k = pl.kernel(
        _lg_fused_body,
        out_type=jax.ShapeDtypeStruct((_NC, _NP, _D), jnp.float32),
        mesh=mesh,
        scratch_types=[
            pltpu.VMEM((2 * ch,), jnp.int32),
            pltpu.VMEM((ch,), jnp.int32),
            pltpu.VMEM((ch,), jnp.int32),
            pltpu.VMEM((ch,), jnp.int32),
            pltpu.VMEM((ch,), jnp.int32),
            pltpu.VMEM((ch,), jnp.float32),
            pltpu.VMEM((ch, _D), jnp.float32),
            pltpu.VMEM_SHARED((_NP, _D), jnp.float32),
            pltpu.SemaphoreType.DMA,
            pltpu.SemaphoreType.DMA,
            pltpu.SemaphoreType.DMA,
            pltpu.SemaphoreType.DMA,
        ],
    )
    out = k(ef, pk, u, v, norm_lg, zeros)
    return out[0] + out[1]


def _epilogue_body(y2a_ref, y2b_ref, d2a_ref, d2b_ref, nrm_ref,
                   Wo_ref, bo_ref, Wm_ref, bm_ref,
                   prs_ref, c2W_ref, c2b_ref, pW_ref, pb_ref,
                   ho_ref, hm_ref, pred_ref):
    nrm = nrm_ref[...]
    ho = jnp.dot(y2a_ref[...] * nrm + d2a_ref[...], Wo_ref[...],
                 preferred_element_type=jnp.float32) + bo_ref[...]
    hm = jnp.dot(y2b_ref[...] * nrm + d2b_ref[...], Wm_ref[...],
                 preferred_element_type=jnp.float32) + bm_ref[...]
    prs = prs_ref[...]
    ho = ho * prs
    hm = hm * prs
    ho_ref[...] = ho
    hm_ref[...] = hm
    z_org = jnp.sum(ho, axis=0, keepdims=True)
    z_meta = jnp.sum(hm, axis=0, keepdims=True)
    Z = jnp.dot(jnp.concatenate([z_meta, z_org], axis=1), c2W_ref[...],
                preferred_element_type=jnp.float32) + c2b_ref[...]
    pred_ref[...] = jnp.dot(Z, pW_ref[...],
                            preferred_element_type=jnp.float32) + pb_ref[...]


def _epilogue(y2a, y2b, d2a, d2b, nrm, Wo, bo, Wm, bm, prs, c2W, c2b,
              pW, pb):
    return pl.pallas_call(
        _epilogue_body,
        out_shape=(
            jax.ShapeDtypeStruct((_N, _D), jnp.float32),
            jax.ShapeDtypeStruct((_N, _D), jnp.float32),
            jax.ShapeDtypeStruct((1, 10), jnp.float32),
        ),
    )(y2a, y2b, d2a, d2b, nrm, Wo, bo, Wm, bm, prs, c2W, c2b, pW, pb)


_EROWS = 2560
_ERPW = _EROWS // _NS


def _pagerank_body(u2_hbm, v2_hbm, invdeg_hbm, out_hbm,
                   u2d, v2d, cu2d, cv2d, pr, invdeg, contrib, zbuf, dbuf,
                   cbuf, acc, contrib_sh, sem_ld, sem_sc):
    c = lax.axis_index("c")
    s = lax.axis_index("s")
    rps = _NP // _NS

    @pl.when(c == 0)
    def _():
        def ldrow(j, carry):
            base = (s * _ERPW + j) * 128
            pltpu.sync_copy(u2_hbm.at[pl.ds(base, 128)], u2d.at[j])
            pltpu.sync_copy(v2_hbm.at[pl.ds(base, 128)], v2d.at[j])
            return carry

        lax.fori_loop(0, _ERPW, ldrow, 0)
        pltpu.sync_copy(invdeg_hbm, invdeg)
        for j in range(40):
            zbuf[pl.ds(j * 16, 16)] = jnp.zeros((16,), jnp.float32)
        init = jnp.full((16,), 1.0 / _N, jnp.float32)
        zero = jnp.zeros((16,), jnp.float32)

        def initp(j, carry):
            pr[pl.ds(j * 16, 16)] = init
            return carry

        def initz(j, carry):
            pr[pl.ds(j * 16, 16)] = zero
            return carry

        lax.fori_loop(0, _N // 16, initp, 0)
        lax.fori_loop(_N // 16, _NP // 16, initz, 0)
        dbuf[pl.ds(0, 16)] = jnp.ones((16,), jnp.float32)

    def body(it, diff2):
        active = jnp.logical_and(c == 0, diff2 >= 1e-12)

        @pl.when(active)
        def _():
            def mkcontrib(j, carry):
                sl = pl.ds(s * rps + j * 16, 16)
                cbuf[pl.ds(j * 16, 16)] = pr[sl] * invdeg[sl]
                return carry

            lax.fori_loop(0, rps // 16, mkcontrib, 0)
            pltpu.sync_copy(cbuf, contrib_sh.at[pl.ds(s * rps, rps)])
            pltpu.sync_copy(zbuf, acc.at[pl.ds(s * rps, rps)])

        plsc.subcore_barrier()

        @pl.when(active)
        def _():
            def srow(b, carry):
                descs = []
                for i in range(8):
                    j = b * 8 + i
                    descs.append(pltpu.async_copy(
                        contrib_sh.at[u2d.at[j]], cu2d.at[j], sem_ld))
                    descs.append(pltpu.async_copy(
                        contrib_sh.at[v2d.at[j]], cv2d.at[j], sem_ld))
                for d in descs:
                    d.wait()
                descs = []
                for i in range(8):
                    j = b * 8 + i
                    descs.append(pltpu.async_copy(
                        cu2d.at[j], acc.at[v2d.at[j]], sem_sc, add=True))
                    descs.append(pltpu.async_copy(
                        cv2d.at[j], acc.at[u2d.at[j]], sem_sc, add=True))
                for d in descs:
                    d.wait()
                return carry

            lax.fori_loop(0, _ERPW // 8, srow, 0)

        plsc.subcore_barrier()

        @pl.when(active)
        def _():
            pltpu.sync_copy(acc, contrib)
            base = jnp.full((16,), 0.15 / _N, jnp.float32)

            def newpr(j, carry):
                sl = pl.ds(j * 16, 16)
                np16 = base + 0.85 * contrib[sl]
                d16 = np16 - pr[sl]
                pr[sl] = np16
                return carry + d16 * d16

            d2 = lax.fori_loop(0, _N // 16, newpr,
                               jnp.zeros((16,), jnp.float32))
            dbuf[pl.ds(0, 16)] = d2

        plsc.subcore_barrier()
        v16 = dbuf[pl.ds(0, 16)]
        tot = v16[0]
        for i in range(1, 16):
            tot = tot + v16[i]
        return tot

    lax.fori_loop(0, 100, body, jnp.float32(1.0))

    @pl.when(c == 0)
    def _():
        pltpu.sync_copy(pr.at[pl.ds(s * rps, rps)],
                        out_hbm.at[pl.ds(s * rps, rps)])


def _pagerank(u2f, v2f, deg_raw):
    invdeg = jnp.zeros((_NP,), jnp.float32).at[:_N].set(
        1.0 / jnp.maximum(deg_raw, 1.0))
    mesh = plsc.VectorSubcoreMesh(**_SC_MESH)
    k = pl.kernel(
        _pagerank_body,
        out_type=jax.ShapeDtypeStruct((_NP,), jnp.float32),
        mesh=mesh,
        scratch_types=[
            pltpu.VMEM((_ERPW, 128), jnp.int32),
            pltpu.VMEM((_ERPW, 128), jnp.int32),
            pltpu.VMEM((_ERPW, 128), jnp.float32),
            pltpu.VMEM((_ERPW, 128), jnp.float32),
            pltpu.VMEM((_NP,), jnp.float32),
            pltpu.VMEM((_NP,), jnp.float32),
            pltpu.VMEM((_NP,), jnp.float32),
            pltpu.VMEM((_NP // _NS,), jnp.float32),
            pltpu.VMEM((16,), jnp.float32),
            pltpu.VMEM((_NP // _NS,), jnp.float32),
            pltpu.VMEM_SHARED((_NP,), jnp.float32),
            pltpu.VMEM_SHARED((_NP,), jnp.float32),
            pltpu.SemaphoreType.DMA,
            pltpu.SemaphoreType.DMA,
        ],
    )
    return k(u2f, v2f, invdeg)[:_N]


def kernel(x, metafeat, edge_index, lg_x, lg_edge_index, batch,
           atom_table, bond_table, meta_W, meta_b,
           W_org, b_org, W_meta, b_meta, W_lg, b_lg,
           W_org1, b_org1, W_meta1, b_meta1, W_lg1, b_lg1,
           cat2_W, cat2_b, pred_W, pred_b):
    u, v = edge_index[0], edge_index[1]
    s_lg0, d_lg0 = lg_edge_index[0], lg_edge_index[1]

    padn = jnp.full((_EROWS * 128 - _E,), _N, jnp.int32)
    u2 = jnp.concatenate([u, padn])
    v2 = jnp.concatenate([v, padn])
    t2 = jnp.concatenate(
        [lg_x, jnp.zeros((_EROWS * 128 - _E,), jnp.int32)])
    lg2 = jnp.concatenate(
        [s_lg0, d_lg0,
         jnp.full((_LGROWS * 128 - 2 * _ELG,), _E, jnp.int32)])

    cdeg, clg = _counts(u2, v2, t2, lg2)
    cnt = cdeg[:_N * 5]
    deg = cdeg[_DEGOFF:_DEGOFF + _N]

    n1, m1, xsa, xsb, da, db, nrm, ivd = _prologue(
        x, metafeat, atom_table, meta_W, meta_b, cnt, bond_table, deg)

    ya, yb = _spmm_dual(xsa, xsb, u, v)

    s_lg, d_lg = lg_edge_index[0], lg_edge_index[1]
    deg_lg = clg[:_E] + 1.0
    norm_lg = jax.lax.rsqrt(deg_lg)
    efp = _edge_build(n1, m1, u, v, lg_x, norm_lg, bond_table)
    eids = jnp.arange(_E, dtype=jnp.int32)
    src_all = jnp.concatenate([s_lg, d_lg, eids])
    dst_all = jnp.concatenate([d_lg, s_lg, eids])
    P = _lg_fused(efp, src_all, dst_all, u, v, norm_lg)[:_N]

    x2a, x2b, d2a, d2b = _mid(ya, yb, da, db, W_org, b_org, W_meta, b_meta,
                              P, W_lg, b_lg, deg, nrm, ivd)

    y2a, y2b = _spmm_dual(x2a, x2b, u, v)

    prs = _pagerank(u2, v2, deg)[:, None]

    ho, hm, pred = _epilogue(
        y2a, y2b, d2a, d2b, nrm, W_org1, b_org1.reshape(1, _D),
        W_meta1, b_meta1.reshape(1, _D), prs, cat2_W,
        cat2_b.reshape(1, _D), pred_W, pred_b.reshape(1, 10))
    return (pred, hm, ho)


# speedup vs baseline: 1.0743x; 1.0743x over previous
import jax
import jax.numpy as jnp
from jax import lax
from jax.experimental import pallas as pl
from jax.experimental.pallas import tpu as pltpu
from jax.experimental.pallas import tpu_sc as plsc

_N = 10000
_E = 320000
_ELG = 640000
_D = 128

_NC = 2
_NS = 16
_NP = 10240

_SC_MESH = dict(core_axis_name="c", subcore_axis_name="s",
                num_cores=_NC, num_subcores=_NS)


def _spmm_dual_body(x2_hbm, u_hbm, v_hbm, zeros_hbm, out_hbm,
                    idx_u, idx_v, idx_ua, idx_va, rows_a, rows_b,
                    acc, sem_a, sem_b):
    c = lax.axis_index("c")
    s = lax.axis_index("s")
    rows_per_sub = _NP // _NS
    pltpu.sync_copy(zeros_hbm.at[pl.ds(s * rows_per_sub, rows_per_sub)],
                    acc.at[pl.ds(s * rows_per_sub, rows_per_sub)])
    plsc.subcore_barrier()

    ch = 80
    chunks = _E // (_NS * ch)
    base0 = s * (_E // _NS)
    coff = c * _NP

    def body(g, carry):
        base = base0 + g * ch
        pltpu.sync_copy(u_hbm.at[pl.ds(base, ch)], idx_u)
        pltpu.sync_copy(v_hbm.at[pl.ds(base, ch)], idx_v)
        for j in range(ch // 16):
            sl = pl.ds(j * 16, 16)
            idx_ua[sl] = idx_u[sl] + coff
            idx_va[sl] = idx_v[sl] + coff
        ga = pltpu.async_copy(x2_hbm.at[idx_ua], rows_a, sem_a)
        gb = pltpu.async_copy(x2_hbm.at[idx_va], rows_b, sem_b)
        ga.wait()
        gb.wait()
        pltpu.sync_copy(rows_a, acc.at[idx_v], add=True)
        pltpu.sync_copy(rows_b, acc.at[idx_u], add=True)
        return carry

    lax.fori_loop(0, chunks, body, 0)
    plsc.subcore_barrier()
    pltpu.sync_copy(acc.at[pl.ds(s * rows_per_sub, rows_per_sub)],
                    out_hbm.at[c, pl.ds(s * rows_per_sub, rows_per_sub)])


def _spmm_dual(xa, xb, u, v):
    x2 = jnp.zeros((2 * _NP, _D), jnp.float32)
    x2 = x2.at[:_N].set(xa).at[_NP:_NP + _N].set(xb)
    zeros = jnp.zeros((_NP, _D), jnp.float32)
    mesh = plsc.VectorSubcoreMesh(**_SC_MESH)
    ch = 80
    k = pl.kernel(
        _spmm_dual_body,
        out_type=jax.ShapeDtypeStruct((_NC, _NP, _D), jnp.float32),
        mesh=mesh,
        scratch_types=[
            pltpu.VMEM((ch,), jnp.int32),
            pltpu.VMEM((ch,), jnp.int32),
            pltpu.VMEM((ch,), jnp.int32),
            pltpu.VMEM((ch,), jnp.int32),
            pltpu.VMEM((ch, _D), jnp.float32),
            pltpu.VMEM((ch, _D), jnp.float32),
            pltpu.VMEM_SHARED((_NP, _D), jnp.float32),
            pltpu.SemaphoreType.DMA,
            pltpu.SemaphoreType.DMA,
        ],
    )
    out = k(x2, u, v, zeros)
    return out[0, :_N], out[1, :_N]


def _elu(t):
    return jnp.where(t > 0, t, jnp.exp(t) - 1.0)


_BLK = 2000


def _prologue_body(x_ref, mfeat_ref, atom_ref, mW_ref, mb_ref, cnt8_ref,
                   bond8_ref, deg_ref,
                   n1_ref, m1_ref, xsa_ref, xsb_ref, da_ref, db_ref,
                   nrm_ref, ivd_ref):
    cols = lax.broadcasted_iota(jnp.int32, (_BLK, _D), 1)
    oh = jnp.where(cols == x_ref[...], 1.0, 0.0)
    n1 = jnp.dot(oh, atom_ref[...], preferred_element_type=jnp.float32)
    m1 = jnp.dot(mfeat_ref[...], mW_ref[...],
                 preferred_element_type=jnp.float32) + mb_ref[...]
    Te = jnp.dot(cnt8_ref[...], bond8_ref[...],
                 preferred_element_type=jnp.float32)
    nf = _elu(n1 * Te)
    mf = _elu(m1 * Te)
    degp = deg_ref[...] + 1.0
    nrm = lax.rsqrt(degp)
    ivd = 1.0 / degp
    n1_ref[...] = n1
    m1_ref[...] = m1
    xsa_ref[...] = nf * nrm
    xsb_ref[...] = mf * nrm
    da_ref[...] = nf * ivd
    db_ref[...] = mf * ivd
    nrm_ref[...] = nrm
    ivd_ref[...] = ivd


def _prologue(x, metafeat, atom_table, meta_W, meta_b, cnt, bond_table, deg):
    x2d = x.reshape(_N, 1).astype(jnp.int32)
    atom128 = jnp.zeros((_D, _D), jnp.float32).at[:119].set(atom_table)
    bond8 = jnp.zeros((8, _D), jnp.float32).at[:5].set(bond_table)
    cnt8 = jnp.zeros((_N, 8), jnp.float32).at[:, :5].set(cnt.reshape(_N, 5))
    nd = jax.ShapeDtypeStruct((_N, _D), jnp.float32)
    n1 = jax.ShapeDtypeStruct((_N, 1), jnp.float32)
    row = lambda w: pl.BlockSpec((_BLK, w), lambda i: (i, 0))
    full = lambda a, b: pl.BlockSpec((a, b), lambda i: (0, 0))
    return pl.pallas_call(
        _prologue_body,
        grid=(_N // _BLK,),
        in_specs=[row(1), row(16), full(_D, _D), full(16, _D), full(1, _D),
                  row(8), full(8, _D), row(1)],
        out_specs=(row(_D), row(_D), row(_D), row(_D), row(_D), row(_D),
                   row(1), row(1)),
        out_shape=(nd, nd, nd, nd, nd, nd, n1, n1),
    )(x2d, metafeat, atom128, meta_W, meta_b.reshape(1, _D), cnt8, bond8,
      deg.reshape(_N, 1))


def _mid_body(ya_ref, yb_ref, da_ref, db_ref, Wo_ref, bo_ref, Wm_ref, bm_ref,
              P_ref, Wlg_ref, blg_ref, deg_ref, nrm_ref, ivd_ref,
              x2a_ref, x2b_ref, d2a_ref, d2b_ref):
    nrm = nrm_ref[...]
    ivd = ivd_ref[...]
    h_org = jnp.dot(ya_ref[...] * nrm + da_ref[...], Wo_ref[...],
                    preferred_element_type=jnp.float32) + bo_ref[...]
    h_meta = jnp.dot(yb_ref[...] * nrm + db_ref[...], Wm_ref[...],
                     preferred_element_type=jnp.float32) + bm_ref[...]
    Th = jnp.dot(P_ref[...], Wlg_ref[...],
                 preferred_element_type=jnp.float32) + deg_ref[...] * blg_ref[...]
    a_o = _elu(h_org * Th)
    a_m = _elu(h_meta * Th)
    x2a_ref[...] = a_o * nrm
    x2b_ref[...] = a_m * nrm
    d2a_ref[...] = a_o * ivd
    d2b_ref[...] = a_m * ivd


def _mid(ya, yb, da, db, W_org, b_org, W_meta, b_meta, P, W_lg, b_lg, deg,
         nrm, ivd):
    nd = jax.ShapeDtypeStruct((_N, _D), jnp.float32)
    row = lambda w: pl.BlockSpec((_BLK, w), lambda i: (i, 0))
    full = lambda a, b: pl.BlockSpec((a, b), lambda i: (0, 0))
    return pl.pallas_call(
        _mid_body,
        grid=(_N // _BLK,),
        in_specs=[row(_D), row(_D), row(_D), row(_D), full(_D, _D),
                  full(1, _D), full(_D, _D), full(1, _D), row(_D),
                  full(_D, _D), full(1, _D), row(1), row(1), row(1)],
        out_specs=(row(_D), row(_D), row(_D), row(_D)),
        out_shape=(nd, nd, nd, nd),
    )(ya, yb, da, db, W_org, b_org.reshape(1, _D), W_meta,
      b_meta.reshape(1, _D), P, W_lg, b_lg.reshape(1, _D),
      deg.reshape(_N, 1), nrm, ivd)


_CNT0 = 61440
_DEGOFF = 51200
_CNT1 = 321536
_LGROWS = 10240


def _counts_body(u2f_hbm, v2f_hbm, t2f_hbm, lg2f_hbm, out0_hbm, out1_hbm,
                 u2d, v2d, t2d, idx4, ones, zbuf, acc, sem_sc):
    c = lax.axis_index("c")
    s = lax.axis_index("s")
    rows = _EROWS // _NS
    lgrows = _LGROWS // _NS

    for j in range(128 // 16):
        sl = pl.ds(j * 16, 16)
        ones[sl] = jnp.ones((16,), jnp.float32)
    for j in range(2048 // 16):
        sl = pl.ds(j * 16, 16)
        zbuf[sl] = jnp.zeros((16,), jnp.float32)

    @pl.when(c == 0)
    def _():
        pltpu.sync_copy(zbuf, acc.at[pl.ds(s * 3840, 2048)])
        pltpu.sync_copy(zbuf.at[pl.ds(0, 1792)],
                        acc.at[pl.ds(s * 3840 + 2048, 1792)])

    @pl.when(c == 1)
    def _():
        def z(j, carry):
            pltpu.sync_copy(zbuf, acc.at[pl.ds(s * 20096 + j * 2048, 2048)])
            return carry

        lax.fori_loop(0, 9, z, 0)
        pltpu.sync_copy(zbuf.at[pl.ds(0, 1664)],
                        acc.at[pl.ds(s * 20096 + 9 * 2048, 1664)])

    plsc.subcore_barrier()

    @pl.when(c == 0)
    def _():
        def ldrow(j, carry):
            base = (s * rows + j) * 128
            pltpu.sync_copy(u2f_hbm.at[pl.ds(base, 128)], u2d.at[j])
            pltpu.sync_copy(v2f_hbm.at[pl.ds(base, 128)], v2d.at[j])
            pltpu.sync_copy(t2f_hbm.at[pl.ds(base, 128)], t2d.at[j])
            return carry

        lax.fori_loop(0, rows, ldrow, 0)

        def row(j, carry):
            for i in range(8):
                sl = pl.ds(i * 16, 16)
                iu = u2d[j, sl]
                iv = v2d[j, sl]
                it = t2d[j, sl]
                idx4[0, sl] = iu * 5 + it
                idx4[1, sl] = iv * 5 + it
                idx4[2, sl] = iu + _DEGOFF
                idx4[3, sl] = iv + _DEGOFF
            for r in range(4):
                pltpu.sync_copy(ones, acc.at[idx4.at[r]], add=True)
            return carry

        lax.fori_loop(0, rows, row, 0)

    @pl.when(c == 1)
    def _():
        def row(j, carry):
            pltpu.sync_copy(lg2f_hbm.at[pl.ds((s * lgrows + j) * 128, 128)],
                            idx4.at[0])
            pltpu.sync_copy(ones, acc.at[idx4.at[0]], add=True)
            return carry

        lax.fori_loop(0, lgrows, row, 0)

    plsc.subcore_barrier()

    @pl.when(c == 0)
    def _():
        pltpu.sync_copy(acc.at[pl.ds(s * 3840, 3840)],
                        out0_hbm.at[pl.ds(s * 3840, 3840)])

    @pl.when(c == 1)
    def _():
        def wb(j, carry):
            pltpu.sync_copy(acc.at[pl.ds(s * 20096 + j * 2048, 2048)],
                            out1_hbm.at[pl.ds(s * 20096 + j * 2048, 2048)])
            return carry

        lax.fori_loop(0, 9, wb, 0)
        pltpu.sync_copy(acc.at[pl.ds(s * 20096 + 9 * 2048, 1664)],
                        out1_hbm.at[pl.ds(s * 20096 + 9 * 2048, 1664)])


def _counts(u2f, v2f, t2f, lg2f):
    mesh = plsc.VectorSubcoreMesh(**_SC_MESH)
    k = pl.kernel(
        _counts_body,
        out_type=(jax.ShapeDtypeStruct((_CNT0,), jnp.float32),
                  jax.ShapeDtypeStruct((_CNT1,), jnp.float32)),
        mesh=mesh,
        scratch_types=[
            pltpu.VMEM((_EROWS // _NS, 128), jnp.int32),
            pltpu.VMEM((_EROWS // _NS, 128), jnp.int32),
            pltpu.VMEM((_EROWS // _NS, 128), jnp.int32),
            pltpu.VMEM((4, 128), jnp.int32),
            pltpu.VMEM((128,), jnp.float32),
            pltpu.VMEM((2048,), jnp.float32),
            pltpu.VMEM_SHARED((_CNT1,), jnp.float32),
            pltpu.SemaphoreType.DMA,
        ],
    )
    return k(u2f, v2f, t2f, lg2f)


def _edge_build_body(n1u_gbl, m1_hbm, pk_hbm, nl_hbm, bt_hbm,
                     ef_hbm,
                     pbuf, nlv, idx_u, idx_v, bt_v, r_n1u, r_n1v, r_m1u,
                     r_m1v, sem0, sem1, sem2, sem3):
    c = lax.axis_index("c")
    s = lax.axis_index("s")
    w = c * _NS + s
    ch = 80
    per_w = _E // (_NC * _NS)
    chunks = per_w // ch
    base0 = w * per_w
    pltpu.sync_copy(bt_hbm, bt_v)

    def body(g, carry):
        base = base0 + g * ch
        pltpu.sync_copy(pk_hbm.at[pl.ds(base * 3, 3 * ch)], pbuf)
        pltpu.sync_copy(nl_hbm.at[pl.ds(base, ch)], nlv)
        for j in range(ch // 16):
            sl = pl.ds(j * 16, 16)
            idx_u[sl] = pbuf[sl]
            idx_v[sl] = pbuf[pl.ds(ch + j * 16, 16)]
        g0 = pltpu.async_copy(n1u_gbl.at[idx_u], r_n1u, sem0)
        g1 = pltpu.async_copy(n1u_gbl.at[idx_v], r_n1v, sem1)
        g2 = pltpu.async_copy(m1_hbm.at[idx_u], r_m1u, sem2)
        g3 = pltpu.async_copy(m1_hbm.at[idx_v], r_m1v, sem3)
        g0.wait()
        g1.wait()
        g2.wait()
        g3.wait()

        def rgrp(g2, carry2):
            tvec = pbuf[pl.ds(2 * ch + g2 * 16, 16)]
            nvec = nlv[pl.ds(g2 * 16, 16)]
            for i in range(16):
                r = g2 * 16 + i
                tr = tvec[i]
                nlr = nvec[i]
                for j in range(_D // 16):
                    sl = pl.ds(j * 16, 16)
                    su = r_n1u[r, sl] + r_n1v[r, sl]
                    sm = r_m1u[r, sl] + r_m1v[r, sl]
                    btj = bt_v[pl.ds(tr * _D + j * 16, 16)]
                    val = btj * su * sm
                    val = jnp.where(val > 0, val, jnp.exp(val) - 1.0)
                    r_n1u[r, sl] = val * nlr
            return carry2

        lax.fori_loop(0, ch // 16, rgrp, 0)
        pltpu.sync_copy(r_n1u, ef_hbm.at[pl.ds(base, ch)])
        return carry

    lax.fori_loop(0, chunks, body, 0)


def _edge_build(n1, m1, u, v, t, norm_lg, bond_table):
    mesh = plsc.VectorSubcoreMesh(**_SC_MESH)
    ch = 80
    pk = jnp.concatenate(
        [u.reshape(-1, ch), v.reshape(-1, ch), t.reshape(-1, ch)],
        axis=1).reshape(-1)
    k = pl.kernel(
        _edge_build_body,
        out_type=jax.ShapeDtypeStruct((_E, _D), jnp.float32),
        mesh=mesh,
        scratch_types=[
            pltpu.VMEM((3 * ch,), jnp.int32),
            pltpu.VMEM((ch,), jnp.float32),
            pltpu.VMEM((ch,), jnp.int32),
            pltpu.VMEM((ch,), jnp.int32),
            pltpu.VMEM((5 * _D,), jnp.float32),
            pltpu.VMEM((ch, _D), jnp.float32),
            pltpu.VMEM((ch, _D), jnp.float32),
            pltpu.VMEM((ch, _D), jnp.float32),
            pltpu.VMEM((ch, _D), jnp.float32),
            pltpu.SemaphoreType.DMA,
            pltpu.SemaphoreType.DMA,
            pltpu.SemaphoreType.DMA,
            pltpu.SemaphoreType.DMA,
        ],
    )
    return k(n1, m1, pk, norm_lg, bond_table.reshape(5 * _D))


def _lg_fused_body(ef_hbm, pk_hbm, u_hbm, v_hbm, nl_hbm, zeros_hbm,
                   out_hbm,
                   pbuf, idx_a, idx_b, idx_u, idx_v, sclv, rows,
                   acc, sem0, sem1, sem2, sem3):
    c = lax.axis_index("c")
    s = lax.axis_index("s")
    w = c * _NS + s
    rows_per_sub = _NP // _NS
    pltpu.sync_copy(zeros_hbm.at[pl.ds(s * rows_per_sub, rows_per_sub)],
                    acc.at[pl.ds(s * rows_per_sub, rows_per_sub)])
    plsc.subcore_barrier()

    ch = 80
    m_tot = 2 * _ELG + _E
    per_w = m_tot // (_NC * _NS)
    chunks = per_w // ch
    base0 = w * per_w

    def body(g, carry):
        base = base0 + g * ch
        pltpu.sync_copy(pk_hbm.at[pl.ds(base * 2, 2 * ch)], pbuf)
        for j in range(ch // 16):
            sl = pl.ds(j * 16, 16)
            idx_a[sl] = pbuf[sl]
            idx_b[sl] = pbuf[pl.ds(ch + j * 16, 16)]
        gr = pltpu.async_copy(ef_hbm.at[idx_a], rows, sem0)
        gs = pltpu.async_copy(nl_hbm.at[idx_b], sclv, sem1)
        gu = pltpu.async_copy(u_hbm.at[idx_b], idx_u, sem2)
        gv = pltpu.async_copy(v_hbm.at[idx_b], idx_v, sem3)
        gr.wait()
        gs.wait()
        gu.wait()
        gv.wait()

        def rgrp(g2, carry2):
            svec = sclv[pl.ds(g2 * 16, 16)]
            for i in range(16):
                r = g2 * 16 + i
                scr = svec[i]
                for j in range(_D // 16):
                    sl = pl.ds(j * 16, 16)
                    rows[r, sl] = rows[r, sl] * scr
            return carry2

        lax.fori_loop(0, ch // 16, rgrp, 0)
        pltpu.sync_copy(rows, acc.at[idx_u], add=True)
        pltpu.sync_copy(rows, acc.at[idx_v], add=True)
        return carry

    lax.fori_loop(0, chunks, body, 0)
    plsc.subcore_barrier()
    pltpu.sync_copy(acc.at[pl.ds(s * rows_per_sub, rows_per_sub)],
                    out_hbm.at[c, pl.ds(s * rows_per_sub, rows_per_sub)])


def _lg_fused(ef, src_all, dst_all, u, v, norm_lg):
    zeros = jnp.zeros((_NP, _D), jnp.float32)
    mesh = plsc.VectorSubcoreMesh(**_SC_MESH)
    ch = 80
    pk = jnp.concatenate(
        [src_all.reshape(-1, ch), dst_all.reshape(-1, ch)],
        axis=1).reshape(-1)
    k = pl.kernel(
        _lg_fused_body,
        out_type=jax.ShapeDtypeStruct((_NC, _NP, _D), jnp.float32),
        mesh=mesh,
        scratch_types=[
            pltpu.VMEM((2 * ch,), jnp.int32),
            pltpu.VMEM((ch,), jnp.int32),
            pltpu.VMEM((ch,), jnp.int32),
            pltpu.VMEM((ch,), jnp.int32),
            pltpu.VMEM((ch,), jnp.int32),
            pltpu.VMEM((ch,), jnp.float32),
            pltpu.VMEM((ch, _D), jnp.float32),
            pltpu.VMEM_SHARED((_NP, _D), jnp.float32),
            pltpu.SemaphoreType.DMA,
            pltpu.SemaphoreType.DMA,
            pltpu.SemaphoreType.DMA,
            pltpu.SemaphoreType.DMA,
        ],
    )
    out = k(ef, pk, u, v, norm_lg, zeros)
    return out[0] + out[1]


def _epilogue_body(y2a_ref, y2b_ref, d2a_ref, d2b_ref, nrm_ref,
                   Wo_ref, bo_ref, Wm_ref, bm_ref,
                   prs_ref, c2W_ref, c2b_ref, pW_ref, pb_ref,
                   ho_ref, hm_ref, pred_ref):
    nrm = nrm_ref[...]
    ho = jnp.dot(y2a_ref[...] * nrm + d2a_ref[...], Wo_ref[...],
                 preferred_element_type=jnp.float32) + bo_ref[...]
    hm = jnp.dot(y2b_ref[...] * nrm + d2b_ref[...], Wm_ref[...],
                 preferred_element_type=jnp.float32) + bm_ref[...]
    prs = prs_ref[...]
    ho = ho * prs
    hm = hm * prs
    ho_ref[...] = ho
    hm_ref[...] = hm
    z_org = jnp.sum(ho, axis=0, keepdims=True)
    z_meta = jnp.sum(hm, axis=0, keepdims=True)
    Z = jnp.dot(jnp.concatenate([z_meta, z_org], axis=1), c2W_ref[...],
                preferred_element_type=jnp.float32) + c2b_ref[...]
    pred_ref[...] = jnp.dot(Z, pW_ref[...],
                            preferred_element_type=jnp.float32) + pb_ref[...]


def _epilogue(y2a, y2b, d2a, d2b, nrm, Wo, bo, Wm, bm, prs, c2W, c2b,
              pW, pb):
    return pl.pallas_call(
        _epilogue_body,
        out_shape=(
            jax.ShapeDtypeStruct((_N, _D), jnp.float32),
            jax.ShapeDtypeStruct((_N, _D), jnp.float32),
            jax.ShapeDtypeStruct((1, 10), jnp.float32),
        ),
    )(y2a, y2b, d2a, d2b, nrm, Wo, bo, Wm, bm, prs, c2W, c2b, pW, pb)


_EROWS = 2560
_ERPW = _EROWS // _NS


def _pagerank_body(u2_hbm, v2_hbm, invdeg_hbm, out_hbm,
                   u2d, v2d, cu2d, cv2d, pr, invdeg, contrib, zbuf, dbuf,
                   cbuf, acc, contrib_sh, sem_ld, sem_sc):
    c = lax.axis_index("c")
    s = lax.axis_index("s")
    rps = _NP // _NS

    @pl.when(c == 0)
    def _():
        def ldrow(j, carry):
            base = (s * _ERPW + j) * 128
            pltpu.sync_copy(u2_hbm.at[pl.ds(base, 128)], u2d.at[j])
            pltpu.sync_copy(v2_hbm.at[pl.ds(base, 128)], v2d.at[j])
            return carry

        lax.fori_loop(0, _ERPW, ldrow, 0)
        pltpu.sync_copy(invdeg_hbm, invdeg)
        for j in range(40):
            zbuf[pl.ds(j * 16, 16)] = jnp.zeros((16,), jnp.float32)
        init = jnp.full((16,), 1.0 / _N, jnp.float32)
        zero = jnp.zeros((16,), jnp.float32)

        def initp(j, carry):
            pr[pl.ds(j * 16, 16)] = init
            return carry

        def initz(j, carry):
            pr[pl.ds(j * 16, 16)] = zero
            return carry

        lax.fori_loop(0, _N // 16, initp, 0)
        lax.fori_loop(_N // 16, _NP // 16, initz, 0)
        dbuf[pl.ds(0, 16)] = jnp.ones((16,), jnp.float32)

    def body(it, diff2):
        active = jnp.logical_and(c == 0, diff2 >= 1e-12)

        @pl.when(active)
        def _():
            def mkcontrib(j, carry):
                sl = pl.ds(s * rps + j * 16, 16)
                cbuf[pl.ds(j * 16, 16)] = pr[sl] * invdeg[sl]
                return carry

            lax.fori_loop(0, rps // 16, mkcontrib, 0)
            pltpu.sync_copy(cbuf, contrib_sh.at[pl.ds(s * rps, rps)])
            pltpu.sync_copy(zbuf, acc.at[pl.ds(s * rps, rps)])

        plsc.subcore_barrier()

        @pl.when(active)
        def _():
            def srow(b, carry):
                descs = []
                for i in range(8):
                    j = b * 8 + i
                    descs.append(pltpu.async_copy(
                        contrib_sh.at[u2d.at[j]], cu2d.at[j], sem_ld))
                    descs.append(pltpu.async_copy(
                        contrib_sh.at[v2d.at[j]], cv2d.at[j], sem_ld))
                for d in descs:
                    d.wait()
                descs = []
                for i in range(8):
                    j = b * 8 + i
                    descs.append(pltpu.async_copy(
                        cu2d.at[j], acc.at[v2d.at[j]], sem_sc, add=True))
                    descs.append(pltpu.async_copy(
                        cv2d.at[j], acc.at[u2d.at[j]], sem_sc, add=True))
                for d in descs:
                    d.wait()
                return carry

            lax.fori_loop(0, _ERPW // 8, srow, 0)

        plsc.subcore_barrier()

        @pl.when(active)
        def _():
            pltpu.sync_copy(acc, contrib)
            base = jnp.full((16,), 0.15 / _N, jnp.float32)

            def newpr(j, carry):
                sl = pl.ds(j * 16, 16)
                np16 = base + 0.85 * contrib[sl]
                d16 = np16 - pr[sl]
                pr[sl] = np16
                return carry + d16 * d16

            d2 = lax.fori_loop(0, _N // 16, newpr,
                               jnp.zeros((16,), jnp.float32))
            dbuf[pl.ds(0, 16)] = d2

        plsc.subcore_barrier()
        v16 = dbuf[pl.ds(0, 16)]
        tot = v16[0]
        for i in range(1, 16):
            tot = tot + v16[i]
        return tot

    lax.fori_loop(0, 100, body, jnp.float32(1.0))

    @pl.when(c == 0)
    def _():
        pltpu.sync_copy(pr.at[pl.ds(s * rps, rps)],
                        out_hbm.at[pl.ds(s * rps, rps)])


def _pagerank(u2f, v2f, deg_raw):
    invdeg = jnp.zeros((_NP,), jnp.float32).at[:_N].set(
        1.0 / jnp.maximum(deg_raw, 1.0))
    mesh = plsc.VectorSubcoreMesh(**_SC_MESH)
    k = pl.kernel(
        _pagerank_body,
        out_type=jax.ShapeDtypeStruct((_NP,), jnp.float32),
        mesh=mesh,
        scratch_types=[
            pltpu.VMEM((_ERPW, 128), jnp.int32),
            pltpu.VMEM((_ERPW, 128), jnp.int32),
            pltpu.VMEM((_ERPW, 128), jnp.float32),
            pltpu.VMEM((_ERPW, 128), jnp.float32),
            pltpu.VMEM((_NP,), jnp.float32),
            pltpu.VMEM((_NP,), jnp.float32),
            pltpu.VMEM((_NP,), jnp.float32),
            pltpu.VMEM((_NP // _NS,), jnp.float32),
            pltpu.VMEM((16,), jnp.float32),
            pltpu.VMEM((_NP // _NS,), jnp.float32),
            pltpu.VMEM_SHARED((_NP,), jnp.float32),
            pltpu.VMEM_SHARED((_NP,), jnp.float32),
            pltpu.SemaphoreType.DMA,
            pltpu.SemaphoreType.DMA,
        ],
    )
    return k(u2f, v2f, invdeg)[:_N]


def kernel(x, metafeat, edge_index, lg_x, lg_edge_index, batch,
           atom_table, bond_table, meta_W, meta_b,
           W_org, b_org, W_meta, b_meta, W_lg, b_lg,
           W_org1, b_org1, W_meta1, b_meta1, W_lg1, b_lg1,
           cat2_W, cat2_b, pred_W, pred_b):
    u, v = edge_index[0], edge_index[1]
    s_lg0, d_lg0 = lg_edge_index[0], lg_edge_index[1]

    padn = jnp.full((_EROWS * 128 - _E,), _N, jnp.int32)
    u2 = jnp.concatenate([u, padn])
    v2 = jnp.concatenate([v, padn])
    t2 = jnp.concatenate(
        [lg_x, jnp.zeros((_EROWS * 128 - _E,), jnp.int32)])
    lg2 = jnp.concatenate(
        [s_lg0, d_lg0,
         jnp.full((_LGROWS * 128 - 2 * _ELG,), _E, jnp.int32)])

    cdeg, clg = _counts(u2, v2, t2, lg2)
    cnt = cdeg[:_N * 5]
    deg = cdeg[_DEGOFF:_DEGOFF + _N]

    n1, m1, xsa, xsb, da, db, nrm, ivd = _prologue(
        x, metafeat, atom_table, meta_W, meta_b, cnt, bond_table, deg)

    ya, yb = _spmm_dual(xsa, xsb, u, v)

    s_lg, d_lg = lg_edge_index[0], lg_edge_index[1]
    deg_lg = clg[:_E] + 1.0
    norm_lg = jax.lax.rsqrt(deg_lg)
    efp = _edge_build(n1, m1, u, v, lg_x, norm_lg, bond_table)
    eids = jnp.arange(_E, dtype=jnp.int32)
    src_all = jnp.concatenate([s_lg, d_lg, eids])
    dst_all = jnp.concatenate([d_lg, s_lg, eids])
    P = _lg_fused(efp, src_all, dst_all, u, v, norm_lg)[:_N]

    x2a, x2b, d2a, d2b = _mid(ya, yb, da, db, W_org, b_org, W_meta, b_meta,
                              P, W_lg, b_lg, deg, nrm, ivd)

    y2a, y2b = _spmm_dual(x2a, x2b, u, v)

    prs = _pagerank(u2, v2, deg)[:, None]

    ho, hm, pred = _epilogue(
        y2a, y2b, d2a, d2b, nrm, W_org1, b_org1.reshape(1, _D),
        W_meta1, b_meta1.reshape(1, _D), prs, cat2_W,
        cat2_b.reshape(1, _D), pred_W, pred_b.reshape(1, 10))
    return (pred, hm, ho)
